# baseline (speedup 1.0000x reference)
import functools

import jax
import jax.numpy as jnp
from jax import lax
from jax.experimental import pallas as pl
from jax.experimental.pallas import tpu as pltpu
from jax.experimental.pallas import tpu_sc as plsc

_BATCH = 16384
_NUM_FEATS = 26
_EMBED_DIM = 32
_OUT_COLS = _EMBED_DIM + _NUM_FEATS - 1

_NC = 2
_NS = 16
_NW = _NC * _NS
_BPW = _BATCH // _NW
_GCHUNK = 128
_NCHUNK = _BPW // _GCHUNK


def _body(x_hbm, table_hbm, out_hbm, x_v, ids_v, emb_v, out_v, sem):
    wid = lax.axis_index("s") * _NC + lax.axis_index("c")
    base = wid * _BPW

    pltpu.sync_copy(x_hbm.at[pl.ds(base, _BPW)], x_v)

    col0 = jnp.zeros((16,), jnp.int32)
    lane = lax.iota(jnp.int32, 16)
    for i in range(_BPW // 16):
        rows = lane + (i * 16)
        ids_f = plsc.load_gather(x_v, [rows, col0])
        ids_v[i // 8, pl.ds((i % 8) * 16, 16)] = ids_f.astype(jnp.int32)

    copies = []
    for j in range(_NCHUNK):
        copies.append(
            pltpu.async_copy(
                table_hbm.at[ids_v.at[j]],
                emb_v.at[pl.ds(j * _GCHUNK, _GCHUNK)],
                sem,
            )
        )
    for c in copies:
        c.wait()

    def row_fn(r, _):
        out_v[r, pl.ds(0, 16)] = emb_v[r, pl.ds(0, 16)]
        out_v[r, pl.ds(16, 16)] = emb_v[r, pl.ds(16, 16)]
        out_v[r, pl.ds(32, 16)] = x_v[r, pl.ds(1, 16)]
        out_v[r, pl.ds(41, 16)] = x_v[r, pl.ds(10, 16)]
        return 0

    lax.fori_loop(0, _BPW, row_fn, 0)

    pltpu.sync_copy(out_v, out_hbm.at[pl.ds(base, _BPW)])


@jax.jit
def kernel(x, embed_table):
    mesh = plsc.VectorSubcoreMesh(core_axis_name="c", subcore_axis_name="s")
    f = functools.partial(
        pl.kernel,
        out_type=jax.ShapeDtypeStruct((_BATCH, _OUT_COLS), jnp.float32),
        mesh=mesh,
        scratch_types=[
            pltpu.VMEM((_BPW, _NUM_FEATS), jnp.float32),
            pltpu.VMEM((_NCHUNK, _GCHUNK), jnp.int32),
            pltpu.VMEM((_BPW, _EMBED_DIM), jnp.float32),
            pltpu.VMEM((_BPW, _OUT_COLS), jnp.float32),
            pltpu.SemaphoreType.DMA,
        ],
        compiler_params=pltpu.CompilerParams(
            needs_layout_passes=False, use_tc_tiling_on_sc=False
        ),
    )(_body)
    return f(x, embed_table)


# speedup vs baseline: 3.2782x; 3.2782x over previous
import functools

import jax
import jax.numpy as jnp
from jax import lax
from jax.experimental import pallas as pl
from jax.experimental.pallas import tpu as pltpu
from jax.experimental.pallas import tpu_sc as plsc

_BATCH = 16384
_NUM_FEATS = 26
_EMBED_DIM = 32
_OUT_COLS = _EMBED_DIM + _NUM_FEATS - 1

_NC = 2
_NS = 16
_NW = _NC * _NS
_BPW = _BATCH // _NW
_CHUNK = 16
_NCHUNKS = _BPW // _CHUNK


def _body(
    xt_hbm, tabt_hbm, out_hbm, idsf_v, ids_v, win_v, rows_v, feat_v, feat2_v, sem
):
    wid = lax.axis_index("s") * _NC + lax.axis_index("c")
    base = wid * _BPW

    pltpu.sync_copy(xt_hbm.at[0].at[pl.ds(base, _BPW)], idsf_v)
    for i in range(_BPW // 16):
        ids_v[pl.ds(i * 16, 16)] = idsf_v[pl.ds(i * 16, 16)].astype(jnp.int32)

    lane = lax.iota(jnp.int32, 16)

    def chunk_fn(c, _):
        idvec = ids_v[pl.ds(c * _CHUNK, _CHUNK)]
        copies = []
        for k in range(_CHUNK):
            t = pl.multiple_of((idvec[k] >> 7) * 128, 128)
            copies.append(
                pltpu.async_copy(
                    tabt_hbm.at[pl.ds(0, _EMBED_DIM), pl.ds(t, 128)],
                    win_v.at[pl.ds(0, _EMBED_DIM), pl.ds(k * 128, 128)],
                    sem,
                )
            )
        for cp in copies:
            cp.wait()
        for k in range(_CHUNK):
            col = jnp.broadcast_to((idvec[k] & 127) + k * 128, (16,))
            j = c * _CHUNK + k
            v0 = plsc.load_gather(win_v, [lane, col])
            v1 = plsc.load_gather(win_v, [lane + 16, col])
            plsc.store_scatter(rows_v, [lane * _BPW + j], v0)
            plsc.store_scatter(rows_v, [(lane + 16) * _BPW + j], v1)
        return 0

    lax.fori_loop(0, _NCHUNKS, chunk_fn, 0)

    for c in range(_EMBED_DIM):
        pltpu.sync_copy(
            rows_v.at[pl.ds(c * _BPW, _BPW)],
            out_hbm.at[c].at[pl.ds(base, _BPW)],
        )

    for j in range(_NUM_FEATS - 1):
        pltpu.sync_copy(xt_hbm.at[1 + j].at[pl.ds(base, _BPW)], feat_v)
        row = _EMBED_DIM + j
        if row == 56:
            for i in range(_BPW // 16):
                feat2_v[0, pl.ds(i * 16, 16)] = feat_v[pl.ds(i * 16, 16)]
            pltpu.sync_copy(
                feat2_v, out_hbm.at[pl.ds(56, 1), pl.ds(base, _BPW)]
            )
        else:
            pltpu.sync_copy(feat_v, out_hbm.at[row].at[pl.ds(base, _BPW)])


@jax.jit
def kernel(x, embed_table):
    mesh = plsc.VectorSubcoreMesh(core_axis_name="c", subcore_axis_name="s")
    f = functools.partial(
        pl.kernel,
        out_type=jax.ShapeDtypeStruct((_OUT_COLS, _BATCH), jnp.float32),
        mesh=mesh,
        scratch_types=[
            pltpu.VMEM((_BPW,), jnp.float32),
            pltpu.VMEM((_BPW,), jnp.int32),
            pltpu.VMEM((_EMBED_DIM, _CHUNK * 128), jnp.float32),
            pltpu.VMEM((_EMBED_DIM * _BPW,), jnp.float32),
            pltpu.VMEM((_BPW,), jnp.float32),
            pltpu.VMEM((1, _BPW), jnp.float32),
            pltpu.SemaphoreType.DMA,
        ],
        compiler_params=pltpu.CompilerParams(
            needs_layout_passes=False,
            use_tc_tiling_on_sc=True,
            disable_bounds_checks=True,
        ),
    )(_body)
    out_t = f(x.T, embed_table.T)
    return out_t.T


# speedup vs baseline: 3.8843x; 1.1849x over previous
import functools

import jax
import jax.numpy as jnp
from jax import lax
from jax.experimental import pallas as pl
from jax.experimental.pallas import tpu as pltpu
from jax.experimental.pallas import tpu_sc as plsc

_BATCH = 16384
_NUM_FEATS = 26
_EMBED_DIM = 32
_OUT_COLS = _EMBED_DIM + _NUM_FEATS - 1

_NC = 2
_NS = 16
_NW = _NC * _NS
_BPW = _BATCH // _NW
_BANK = 8
_CHUNK = 2 * _BANK
_NCHUNKS = _BPW // _CHUNK


def _body(
    xt_hbm, tabt_hbm, out_hbm,
    idsf_v, ids_v, win_v, rows_v, feats_v, feat2_v,
    sem_a, sem_b, sem_f, sem_w,
):
    wid = lax.axis_index("s") * _NC + lax.axis_index("c")
    base = wid * _BPW
    lane = lax.iota(jnp.int32, 16)
    banks = (sem_a, sem_b)

    feat_reads = []
    for j in range(_NUM_FEATS - 1):
        feat_reads.append(
            pltpu.async_copy(
                xt_hbm.at[1 + j].at[pl.ds(base, _BPW)],
                feats_v.at[pl.ds(j * _BPW, _BPW)],
                sem_f,
            )
        )

    pltpu.sync_copy(xt_hbm.at[0].at[pl.ds(base, _BPW)], idsf_v)
    for i in range(_BPW // 16):
        ids_v[pl.ds(i * 16, 16)] = idsf_v[pl.ds(i * 16, 16)].astype(jnp.int32)

    def slot_ref(h, k):
        return win_v.at[pl.ds(0, _EMBED_DIM), pl.ds((h * _BANK + k) * 128, 128)]

    dummy_src = tabt_hbm.at[pl.ds(0, _EMBED_DIM), pl.ds(0, 128)]

    def fire(idv, h, k):
        t = pl.multiple_of((idv >> 7) * 128, 128)
        pltpu.async_copy(
            tabt_hbm.at[pl.ds(0, _EMBED_DIM), pl.ds(t, 128)],
            slot_ref(h, k),
            banks[h],
        )

    def extract(idv, h, k, j):
        col = jnp.broadcast_to((idv & 127) + (h * _BANK + k) * 128, (16,))
        v0 = plsc.load_gather(win_v, [lane, col])
        v1 = plsc.load_gather(win_v, [lane + 16, col])
        plsc.store_scatter(rows_v, [lane * _BPW + j], v0)
        plsc.store_scatter(rows_v, [(lane + 16) * _BPW + j], v1)

    def chunk_fn(c, _):
        cm1 = lax.max(c - 1, 0)
        idvec_cur = ids_v[pl.ds(c * _CHUNK, _CHUNK)]
        idvec_prev = ids_v[pl.ds(cm1 * _CHUNK, _CHUNK)]
        for h in (0, 1):
            @pl.when(c > 0)
            def _(h=h):
                for k in range(_BANK):
                    pltpu.make_async_copy(dummy_src, slot_ref(h, k), banks[h]).wait()
                for k in range(_BANK):
                    extract(
                        idvec_prev[h * _BANK + k], h, k,
                        cm1 * _CHUNK + h * _BANK + k,
                    )
            for k in range(_BANK):
                fire(idvec_cur[h * _BANK + k], h, k)
        return 0

    lax.fori_loop(0, _NCHUNKS, chunk_fn, 0)

    idvec_last = ids_v[pl.ds((_NCHUNKS - 1) * _CHUNK, _CHUNK)]
    for h in (0, 1):
        for k in range(_BANK):
            pltpu.make_async_copy(dummy_src, slot_ref(h, k), banks[h]).wait()
        for k in range(_BANK):
            extract(
                idvec_last[h * _BANK + k], h, k,
                (_NCHUNKS - 1) * _CHUNK + h * _BANK + k,
            )

    writes = []
    for c in range(_EMBED_DIM):
        writes.append(
            pltpu.async_copy(
                rows_v.at[pl.ds(c * _BPW, _BPW)],
                out_hbm.at[c].at[pl.ds(base, _BPW)],
                sem_w,
            )
        )

    for cp in feat_reads:
        cp.wait()
    for j in range(_NUM_FEATS - 1):
        row = _EMBED_DIM + j
        if row == 56:
            for i in range(_BPW // 16):
                feat2_v[0, pl.ds(i * 16, 16)] = feats_v[
                    pl.ds(j * _BPW + i * 16, 16)
                ]
            writes.append(
                pltpu.async_copy(
                    feat2_v, out_hbm.at[pl.ds(56, 1), pl.ds(base, _BPW)], sem_w
                )
            )
        else:
            writes.append(
                pltpu.async_copy(
                    feats_v.at[pl.ds(j * _BPW, _BPW)],
                    out_hbm.at[row].at[pl.ds(base, _BPW)],
                    sem_w,
                )
            )
    for cp in writes:
        cp.wait()


@jax.jit
def kernel(x, embed_table):
    mesh = plsc.VectorSubcoreMesh(core_axis_name="c", subcore_axis_name="s")
    f = functools.partial(
        pl.kernel,
        out_type=jax.ShapeDtypeStruct((_OUT_COLS, _BATCH), jnp.float32),
        mesh=mesh,
        scratch_types=[
            pltpu.VMEM((_BPW,), jnp.float32),
            pltpu.VMEM((_BPW,), jnp.int32),
            pltpu.VMEM((_EMBED_DIM, _CHUNK * 128), jnp.float32),
            pltpu.VMEM((_EMBED_DIM * _BPW,), jnp.float32),
            pltpu.VMEM(((_NUM_FEATS - 1) * _BPW,), jnp.float32),
            pltpu.VMEM((1, _BPW), jnp.float32),
            pltpu.SemaphoreType.DMA,
            pltpu.SemaphoreType.DMA,
            pltpu.SemaphoreType.DMA,
            pltpu.SemaphoreType.DMA,
        ],
        compiler_params=pltpu.CompilerParams(
            needs_layout_passes=False,
            use_tc_tiling_on_sc=True,
            disable_bounds_checks=True,
        ),
    )(_body)
    out_t = f(x.T, embed_table.T)
    return out_t.T


# speedup vs baseline: 4.2560x; 1.0957x over previous
import functools

import jax
import jax.numpy as jnp
from jax import lax
from jax.experimental import pallas as pl
from jax.experimental.pallas import tpu as pltpu
from jax.experimental.pallas import tpu_sc as plsc

_BATCH = 16384
_NUM_FEATS = 26
_EMBED_DIM = 32
_OUT_COLS = _EMBED_DIM + _NUM_FEATS - 1

_NC = 2
_NS = 16
_NW = _NC * _NS
_BPW = _BATCH // _NW
_BANK = 8
_CHUNK = 2 * _BANK
_NCHUNKS = _BPW // _CHUNK


def _body(
    xt_hbm, tabt_hbm, out_hbm,
    idsf_v, ids_v, win_v, rows_v, feats_v, feat2_v,
    slot_sems, sem_f, sem_w,
):
    wid = lax.axis_index("s") * _NC + lax.axis_index("c")
    base = wid * _BPW
    lane = lax.iota(jnp.int32, 16)

    feat_reads = []
    for j in range(_NUM_FEATS - 1):
        feat_reads.append(
            pltpu.async_copy(
                xt_hbm.at[1 + j].at[pl.ds(base, _BPW)],
                feats_v.at[pl.ds(j * _BPW, _BPW)],
                sem_f,
            )
        )

    pltpu.sync_copy(xt_hbm.at[0].at[pl.ds(base, _BPW)], idsf_v)
    for i in range(_BPW // 16):
        ids_v[pl.ds(i * 16, 16)] = idsf_v[pl.ds(i * 16, 16)].astype(jnp.int32)

    def slot_ref(k):
        return win_v.at[pl.ds(0, _EMBED_DIM), pl.ds(k * 128, 128)]

    dummy_src = tabt_hbm.at[pl.ds(0, _EMBED_DIM), pl.ds(0, 128)]

    def fire(idv, k):
        t = pl.multiple_of((idv >> 7) * 128, 128)
        pltpu.async_copy(
            tabt_hbm.at[pl.ds(0, _EMBED_DIM), pl.ds(t, 128)],
            slot_ref(k),
            slot_sems.at[k],
        )

    def extract(idv, k, j):
        col = jnp.broadcast_to((idv & 127) + k * 128, (16,))
        v0 = plsc.load_gather(win_v, [lane, col])
        v1 = plsc.load_gather(win_v, [lane + 16, col])
        plsc.store_scatter(rows_v, [lane * _BPW + j], v0)
        plsc.store_scatter(rows_v, [(lane + 16) * _BPW + j], v1)

    def chunk_fn(c, _):
        cm1 = lax.max(c - 1, 0)
        idvec_cur = ids_v[pl.ds(c * _CHUNK, _CHUNK)]
        idvec_prev = ids_v[pl.ds(cm1 * _CHUNK, _CHUNK)]
        for k in range(_CHUNK):
            @pl.when(c > 0)
            def _(k=k):
                pltpu.make_async_copy(
                    dummy_src, slot_ref(k), slot_sems.at[k]
                ).wait()
                extract(idvec_prev[k], k, cm1 * _CHUNK + k)
            fire(idvec_cur[k], k)
        return 0

    lax.fori_loop(0, _NCHUNKS, chunk_fn, 0)

    idvec_last = ids_v[pl.ds((_NCHUNKS - 1) * _CHUNK, _CHUNK)]
    for k in range(_CHUNK):
        pltpu.make_async_copy(dummy_src, slot_ref(k), slot_sems.at[k]).wait()
        extract(idvec_last[k], k, (_NCHUNKS - 1) * _CHUNK + k)

    writes = []
    for c in range(_EMBED_DIM):
        writes.append(
            pltpu.async_copy(
                rows_v.at[pl.ds(c * _BPW, _BPW)],
                out_hbm.at[c].at[pl.ds(base, _BPW)],
                sem_w,
            )
        )

    for cp in feat_reads:
        cp.wait()
    for j in range(_NUM_FEATS - 1):
        row = _EMBED_DIM + j
        if row == 56:
            for i in range(_BPW // 16):
                feat2_v[0, pl.ds(i * 16, 16)] = feats_v[
                    pl.ds(j * _BPW + i * 16, 16)
                ]
            writes.append(
                pltpu.async_copy(
                    feat2_v, out_hbm.at[pl.ds(56, 1), pl.ds(base, _BPW)], sem_w
                )
            )
        else:
            writes.append(
                pltpu.async_copy(
                    feats_v.at[pl.ds(j * _BPW, _BPW)],
                    out_hbm.at[row].at[pl.ds(base, _BPW)],
                    sem_w,
                )
            )
    for cp in writes:
        cp.wait()


@jax.jit
def kernel(x, embed_table):
    mesh = plsc.VectorSubcoreMesh(core_axis_name="c", subcore_axis_name="s")
    f = functools.partial(
        pl.kernel,
        out_type=jax.ShapeDtypeStruct((_OUT_COLS, _BATCH), jnp.float32),
        mesh=mesh,
        scratch_types=[
            pltpu.VMEM((_BPW,), jnp.float32),
            pltpu.VMEM((_BPW,), jnp.int32),
            pltpu.VMEM((_EMBED_DIM, _CHUNK * 128), jnp.float32),
            pltpu.VMEM((_EMBED_DIM * _BPW,), jnp.float32),
            pltpu.VMEM(((_NUM_FEATS - 1) * _BPW,), jnp.float32),
            pltpu.VMEM((1, _BPW), jnp.float32),
            pltpu.SemaphoreType.DMA((_CHUNK,)),
            pltpu.SemaphoreType.DMA,
            pltpu.SemaphoreType.DMA,
        ],
        compiler_params=pltpu.CompilerParams(
            needs_layout_passes=False,
            use_tc_tiling_on_sc=True,
            disable_bounds_checks=True,
        ),
    )(_body)
    out_t = f(x.T, embed_table.T)
    return out_t.T
